# Initial kernel scaffold; baseline (speedup 1.0000x reference)
import jax
import jax.numpy as jnp
from jax.experimental import pallas as pl


def kernel(input_ids, cluster_assign, centroids, offsets):
    raise NotImplementedError("write your pallas kernel here")


# speedup vs baseline: 1.9335x; 1.9335x over previous
import functools

import jax
import jax.numpy as jnp
from jax import lax
from jax.experimental import pallas as pl
from jax.experimental.pallas import tpu as pltpu
from jax.experimental.pallas import tpu_sc as plsc

D = 64
L = 16
NC, NS = 2, 16
NW = NC * NS
G = 128
GPC = 5
K = G * GPC


@functools.lru_cache(maxsize=None)
def _build(ntok):
    n_per_w = ntok // NW
    nchunks = n_per_w // K
    assert n_per_w % K == 0
    mesh = plsc.VectorSubcoreMesh(
        core_axis_name="c", subcore_axis_name="s", num_cores=NC, num_subcores=NS
    )

    @functools.partial(
        pl.kernel,
        out_type=jax.ShapeDtypeStruct((ntok, D), jnp.float32),
        mesh=mesh,
        scratch_types=[
            pltpu.VMEM((GPC, G), jnp.int32),
            pltpu.VMEM((GPC, G), jnp.int32),
            pltpu.VMEM((K, D), jnp.float32),
            pltpu.VMEM((K, D), jnp.float32),
            pltpu.SemaphoreType.DMA,
        ],
        compiler_params=pltpu.CompilerParams(use_tc_tiling_on_sc=False),
    )
    def sc_kernel(ids_hbm, ca_hbm, cent_hbm, off_hbm, out_hbm,
                  ids_v, cids_v, acc_v, cen_v, sem):
        wid = lax.axis_index("s") * NC + lax.axis_index("c")
        wbase = wid * n_per_w

        @pl.loop(0, nchunks)
        def _chunk(c):
            base = wbase + c * K
            for j in range(GPC):
                pltpu.sync_copy(ids_hbm.at[pl.ds(base + j * G, G)], ids_v.at[j])
            descs = [
                pltpu.async_copy(ca_hbm.at[ids_v.at[j]], cids_v.at[j], sem)
                for j in range(GPC)
            ] + [
                pltpu.async_copy(
                    off_hbm.at[ids_v.at[j]], acc_v.at[pl.ds(j * G, G)], sem
                )
                for j in range(GPC)
            ]
            for dsc in descs:
                dsc.wait()
            descs = [
                pltpu.async_copy(
                    cent_hbm.at[cids_v.at[j]], cen_v.at[pl.ds(j * G, G)], sem
                )
                for j in range(GPC)
            ]
            for dsc in descs:
                dsc.wait()

            @pl.loop(0, K)
            def _add(t):
                for d in range(D // L):
                    sl = pl.ds(d * L, L)
                    acc_v[t, sl] = acc_v[t, sl] + cen_v[t, sl]

            pltpu.sync_copy(acc_v, out_hbm.at[pl.ds(base, K)])

    return sc_kernel


def kernel(input_ids, cluster_assign, centroids, offsets):
    b, t = input_ids.shape
    ids = input_ids.reshape(-1)
    out = _build(ids.shape[0])(ids, cluster_assign, centroids, offsets)
    return out.reshape(b, t, D)
